# baseline (speedup 1.0000x reference)
import jax
import jax.numpy as jnp
from jax.experimental import pallas as pl

_K = 1024
_C = 64


def _match_body(x_ref, bank_ref, recon_ref, idx_ref):
    B = x_ref.shape[0]
    hw = x_ref.shape[2]
    bank = bank_ref[...]
    nsq = jnp.sum(bank * bank, axis=1, keepdims=True)
    pn = bank / jnp.maximum(jnp.sqrt(nsq), 1e-12)
    cols = []
    for b in range(B):
        xb = x_ref[b]
        xsq = jnp.sum(xb * xb, axis=0, keepdims=True)
        cols.append(xb / jnp.maximum(jnp.sqrt(xsq), 1e-12))
    xn = jnp.concatenate(cols, axis=1)
    s = jnp.dot(pn, xn, preferred_element_type=jnp.float32)
    idx = jnp.argmax(s, axis=0)[None, :]
    iota = jax.lax.broadcasted_iota(jnp.int32, (_K, B * hw), 0)
    onehot = (iota == idx).astype(jnp.float32)
    recon = jax.lax.dot_general(
        bank, onehot, (((0,), (0,)), ((), ())),
        preferred_element_type=jnp.float32)
    H = recon_ref.shape[2]
    W = recon_ref.shape[3]
    for b in range(B):
        recon_ref[b] = recon[:, b * hw:(b + 1) * hw].reshape(_C, H, W)
        idx_ref[b] = idx[0, b * hw:(b + 1) * hw]


def kernel(x, prototype_bank):
    B, C, H, W = x.shape
    HW = H * W
    x3 = x.reshape(B, C, HW)
    recon, idx = pl.pallas_call(
        _match_body,
        out_shape=[
            jax.ShapeDtypeStruct((B, C, H, W), jnp.float32),
            jax.ShapeDtypeStruct((B, HW), jnp.int32),
        ],
    )(x3, prototype_bank)
    return recon, idx


# speedup vs baseline: 1.2444x; 1.2444x over previous
import jax
import jax.numpy as jnp
from jax.experimental import pallas as pl

_K = 1024
_C = 64


def _match_body(x_ref, bank_ref, recon_ref, idx_ref):
    B = x_ref.shape[0]
    hw = x_ref.shape[2]
    bank = bank_ref[...]
    nsq = jnp.sum(bank * bank, axis=1, keepdims=True)
    pn = bank / jnp.maximum(jnp.sqrt(nsq), 1e-12)
    cols = []
    for b in range(B):
        xb = x_ref[b]
        xsq = jnp.sum(xb * xb, axis=0, keepdims=True)
        cols.append(xb / jnp.maximum(jnp.sqrt(xsq), 1e-12))
    xn = jnp.concatenate(cols, axis=1)
    s = jnp.dot(pn, xn, preferred_element_type=jnp.float32)
    idx = jnp.argmax(s, axis=0)[None, :]
    iota = jax.lax.broadcasted_iota(jnp.int32, (_K, B * hw), 0)
    onehot = (iota == idx).astype(jnp.float32)
    recon = jax.lax.dot_general(
        bank, onehot, (((0,), (0,)), ((), ())),
        preferred_element_type=jnp.float32)
    for b in range(B):
        recon_ref[b] = recon[:, b * hw:(b + 1) * hw]
        idx_ref[b] = idx[:, b * hw:(b + 1) * hw]


def kernel(x, prototype_bank):
    B, C, H, W = x.shape
    HW = H * W
    x3 = x.reshape(B, C, HW)
    recon3, idx3 = pl.pallas_call(
        _match_body,
        out_shape=[
            jax.ShapeDtypeStruct((B, C, HW), jnp.float32),
            jax.ShapeDtypeStruct((B, 1, HW), jnp.int32),
        ],
    )(x3, prototype_bank)
    return recon3.reshape(B, C, H, W), idx3.reshape(B, HW)
